# baseline (speedup 1.0000x reference)
import functools

import jax
import jax.numpy as jnp
from jax import lax
from jax.experimental import pallas as pl
from jax.experimental.pallas import tpu as pltpu
from jax.experimental.pallas import tpu_sc as plsc

BATCH = 4096
DIM = 64
NUM_CORES = 2
NUM_SUBCORES = 16
NUM_WORKERS = NUM_CORES * NUM_SUBCORES
ROWS_PER_WORKER = BATCH // NUM_WORKERS
GROUPS = ROWS_PER_WORKER // 16
E_SPLIT = 500000

_mesh = plsc.VectorSubcoreMesh(core_axis_name="c", subcore_axis_name="s")


@functools.partial(
    pl.kernel,
    out_type=jax.ShapeDtypeStruct((BATCH,), jnp.float32),
    mesh=_mesh,
    scratch_types=[
        pltpu.VMEM((ROWS_PER_WORKER,), jnp.int32),
        pltpu.VMEM((ROWS_PER_WORKER,), jnp.int32),
        pltpu.VMEM((ROWS_PER_WORKER,), jnp.int32),
        pltpu.VMEM((ROWS_PER_WORKER, DIM), jnp.float32),
        pltpu.VMEM((ROWS_PER_WORKER, DIM), jnp.float32),
        pltpu.VMEM((ROWS_PER_WORKER, DIM), jnp.float32),
        pltpu.VMEM((ROWS_PER_WORKER,), jnp.float32),
        pltpu.SemaphoreType.DMA,
        pltpu.SemaphoreType.DMA,
    ],
)
def _distmult_sc(s_hbm, r_hbm, o_hbm, e_hbm, rel_hbm, out_hbm,
                 si, ri, oi, se, re, oe, ov, sem_a, sem_b):
    wid = lax.axis_index("s") * NUM_CORES + lax.axis_index("c")
    base = wid * ROWS_PER_WORKER

    pltpu.sync_copy(s_hbm.at[pl.ds(base, ROWS_PER_WORKER)], si)
    pltpu.sync_copy(r_hbm.at[pl.ds(base, ROWS_PER_WORKER)], ri)
    pltpu.sync_copy(o_hbm.at[pl.ds(base, ROWS_PER_WORKER)], oi)

    half = jnp.int32(E_SPLIT)
    wave = ROWS_PER_WORKER // 2

    def make_fire(sem):
        def fire(g, carry):
            sv = si[pl.ds(g * 16, 16)]
            rv = ri[pl.ds(g * 16, 16)]
            owv = oi[pl.ds(g * 16, 16)]
            for l in range(16):
                row = g * 16 + l
                sa = (sv[l] >= half).astype(jnp.int32)
                oa = (owv[l] >= half).astype(jnp.int32)
                pltpu.async_copy(e_hbm.at[sa, sv[l] - sa * half],
                                 se.at[row], sem)
                pltpu.async_copy(rel_hbm.at[rv[l]], re.at[row], sem)
                pltpu.async_copy(e_hbm.at[oa, owv[l] - oa * half],
                                 oe.at[row], sem)
            return carry
        return fire

    lax.fori_loop(0, GROUPS // 2, make_fire(sem_a), 0)
    lax.fori_loop(GROUPS // 2, GROUPS, make_fire(sem_b), 0)

    lanes = lax.iota(jnp.int32, 16)
    dnums = lax.GatherDimensionNumbers(
        offset_dims=(), collapsed_slice_dims=(0,), start_index_map=(0,))

    def lane_perm(x, idx):
        return lax.gather(x, idx[:, None], dnums, slice_sizes=(1,),
                          mode=lax.GatherScatterMode.PROMISE_IN_BOUNDS)

    def group_body(g, carry):
        res = jnp.zeros((16,), jnp.float32)
        for l in range(16):
            row = g * 16 + l
            acc = (se[row, pl.ds(0, 16)]
                   * re[row, pl.ds(0, 16)]
                   * oe[row, pl.ds(0, 16)])
            for c in range(1, DIM // 16):
                acc = acc + (se[row, pl.ds(c * 16, 16)]
                             * re[row, pl.ds(c * 16, 16)]
                             * oe[row, pl.ds(c * 16, 16)])
            for step in (1, 2, 4, 8):
                acc = acc + lane_perm(acc, lanes ^ step)
            res = jnp.where(lanes == l, acc, res)
        ov[pl.ds(g * 16, 16)] = res
        return carry

    pltpu.make_async_copy(e_hbm.at[0].at[pl.ds(0, wave)],
                          se.at[pl.ds(0, wave)], sem_a).wait()
    pltpu.make_async_copy(rel_hbm.at[pl.ds(0, wave)],
                          re.at[pl.ds(0, wave)], sem_a).wait()
    pltpu.make_async_copy(e_hbm.at[0].at[pl.ds(0, wave)],
                          oe.at[pl.ds(0, wave)], sem_a).wait()
    lax.fori_loop(0, GROUPS // 2, group_body, 0)

    pltpu.make_async_copy(e_hbm.at[0].at[pl.ds(0, wave)],
                          se.at[pl.ds(wave, wave)], sem_b).wait()
    pltpu.make_async_copy(rel_hbm.at[pl.ds(0, wave)],
                          re.at[pl.ds(wave, wave)], sem_b).wait()
    pltpu.make_async_copy(e_hbm.at[0].at[pl.ds(0, wave)],
                          oe.at[pl.ds(wave, wave)], sem_b).wait()
    lax.fori_loop(GROUPS // 2, GROUPS, group_body, 0)

    pltpu.sync_copy(ov, out_hbm.at[pl.ds(base, ROWS_PER_WORKER)])


def kernel(s, r, o, E, R):
    s1 = s.reshape(-1).astype(jnp.int32)
    r1 = r.reshape(-1).astype(jnp.int32)
    o1 = o.reshape(-1).astype(jnp.int32)
    E3 = E.reshape(2, E_SPLIT, DIM)
    out = _distmult_sc(s1, r1, o1, E3, R)
    return out.reshape(BATCH, 1)


# speedup vs baseline: 1.0101x; 1.0101x over previous
import functools

import jax
import jax.numpy as jnp
from jax import lax
from jax.experimental import pallas as pl
from jax.experimental.pallas import tpu as pltpu
from jax.experimental.pallas import tpu_sc as plsc

BATCH = 4096
DIM = 64
NUM_CORES = 2
NUM_SUBCORES = 16
NUM_WORKERS = NUM_CORES * NUM_SUBCORES
ROWS_PER_WORKER = BATCH // NUM_WORKERS
GROUPS = ROWS_PER_WORKER // 16
E_SPLIT = 500000

_mesh = plsc.VectorSubcoreMesh(core_axis_name="c", subcore_axis_name="s")


@functools.partial(
    pl.kernel,
    out_type=jax.ShapeDtypeStruct((BATCH,), jnp.float32),
    mesh=_mesh,
    scratch_types=[
        pltpu.VMEM((ROWS_PER_WORKER,), jnp.int32),
        pltpu.VMEM((ROWS_PER_WORKER,), jnp.int32),
        pltpu.VMEM((ROWS_PER_WORKER,), jnp.int32),
        pltpu.VMEM((ROWS_PER_WORKER, DIM), jnp.float32),
        pltpu.VMEM((ROWS_PER_WORKER, DIM), jnp.float32),
        pltpu.VMEM((ROWS_PER_WORKER, DIM), jnp.float32),
        pltpu.VMEM((ROWS_PER_WORKER,), jnp.float32),
        pltpu.SemaphoreType.DMA,
    ],
)
def _distmult_sc(s_hbm, r_hbm, o_hbm, e_hbm, rel_hbm, out_hbm,
                 si, ri, oi, se, re, oe, ov, sem):
    wid = lax.axis_index("s") * NUM_CORES + lax.axis_index("c")
    base = wid * ROWS_PER_WORKER

    pltpu.sync_copy(s_hbm.at[pl.ds(base, ROWS_PER_WORKER)], si)
    pltpu.sync_copy(r_hbm.at[pl.ds(base, ROWS_PER_WORKER)], ri)
    pltpu.sync_copy(o_hbm.at[pl.ds(base, ROWS_PER_WORKER)], oi)

    half = jnp.int32(E_SPLIT)

    def fire(g, carry):
        sv = si[pl.ds(g * 16, 16)]
        rv = ri[pl.ds(g * 16, 16)]
        owv = oi[pl.ds(g * 16, 16)]
        for l in range(16):
            row = g * 16 + l
            sa = (sv[l] >= half).astype(jnp.int32)
            oa = (owv[l] >= half).astype(jnp.int32)
            pltpu.async_copy(e_hbm.at[sa, sv[l] - sa * half],
                             se.at[row], sem)
            pltpu.async_copy(rel_hbm.at[rv[l]], re.at[row], sem)
            pltpu.async_copy(e_hbm.at[oa, owv[l] - oa * half],
                             oe.at[row], sem)
        return carry

    lax.fori_loop(0, GROUPS, fire, 0)

    lanes = lax.iota(jnp.int32, 16)
    dnums = lax.GatherDimensionNumbers(
        offset_dims=(), collapsed_slice_dims=(0,), start_index_map=(0,))

    def lane_perm(x, idx):
        return lax.gather(x, idx[:, None], dnums, slice_sizes=(1,),
                          mode=lax.GatherScatterMode.PROMISE_IN_BOUNDS)

    def group_body(g, carry):
        res = jnp.zeros((16,), jnp.float32)
        for l in range(16):
            row = g * 16 + l
            acc = (se[row, pl.ds(0, 16)]
                   * re[row, pl.ds(0, 16)]
                   * oe[row, pl.ds(0, 16)])
            for c in range(1, DIM // 16):
                acc = acc + (se[row, pl.ds(c * 16, 16)]
                             * re[row, pl.ds(c * 16, 16)]
                             * oe[row, pl.ds(c * 16, 16)])
            for step in (1, 2, 4, 8):
                acc = acc + lane_perm(acc, lanes ^ step)
            res = jnp.where(lanes == l, acc, res)
        ov[pl.ds(g * 16, 16)] = res
        return carry

    pltpu.make_async_copy(e_hbm.at[0].at[pl.ds(0, ROWS_PER_WORKER)],
                          se, sem).wait()
    pltpu.make_async_copy(rel_hbm.at[pl.ds(0, ROWS_PER_WORKER)],
                          re, sem).wait()
    pltpu.make_async_copy(e_hbm.at[0].at[pl.ds(0, ROWS_PER_WORKER)],
                          oe, sem).wait()
    lax.fori_loop(0, GROUPS, group_body, 0)

    pltpu.sync_copy(ov, out_hbm.at[pl.ds(base, ROWS_PER_WORKER)])


def kernel(s, r, o, E, R):
    s1 = s.reshape(-1).astype(jnp.int32)
    r1 = r.reshape(-1).astype(jnp.int32)
    o1 = o.reshape(-1).astype(jnp.int32)
    E3 = E.reshape(2, E_SPLIT, DIM)
    out = _distmult_sc(s1, r1, o1, E3, R)
    return out.reshape(BATCH, 1)
